# initial kernel scaffold (unmeasured)
import jax
import jax.numpy as jnp
from jax import lax
from jax.experimental import pallas as pl
from jax.experimental.pallas import tpu as pltpu

N_DEV = 4


def kernel(t, W):
    m_per, k = t.shape
    n = W.shape[1]
    m_c = m_per // N_DEV

    def body(t_ref, w_ref, out_ref, acc_ref, recv_ref, send_sems, recv_sems):
        d = lax.axis_index("i")
        left = (d - 1) % N_DEV
        right = (d + 1) % N_DEV

        barrier_sem = pltpu.get_barrier_semaphore()
        for nbr in (left, right):
            pl.semaphore_signal(
                barrier_sem, inc=1,
                device_id=(nbr,), device_id_type=pl.DeviceIdType.MESH,
            )
        pl.semaphore_wait(barrier_sem, 2)

        def t_chunk(c):
            return t_ref[pl.ds(c * m_c, m_c), :]

        for h in range(N_DEV - 1):
            c_send = (d - 1 - h) % N_DEV
            c_recv = (d - 2 - h) % N_DEV
            if h == 0:
                src = t_ref.at[pl.ds(c_send * m_c, m_c), :]
            else:
                src = acc_ref.at[:, :]
            rdma = pltpu.make_async_remote_copy(
                src_ref=src,
                dst_ref=recv_ref.at[h],
                send_sem=send_sems.at[h],
                recv_sem=recv_sems.at[h],
                device_id=(right,),
                device_id_type=pl.DeviceIdType.MESH,
            )
            rdma.start()
            rdma.wait()
            if h < N_DEV - 2:
                acc_ref[:, :] = recv_ref[h] + t_chunk(c_recv)

        s_chunk = recv_ref[N_DEV - 2] + t_chunk(d)

        y = lax.dot_general(
            s_chunk.astype(jnp.bfloat16),
            w_ref[:, :].astype(jnp.bfloat16),
            (((1,), (0,)), ((), ())),
            preferred_element_type=jnp.float32,
        )
        out_ref[pl.ds(d * m_c, m_c), :] = y

        for h in range(N_DEV - 1):
            c_send = (d - h) % N_DEV
            c_recv = (d - 1 - h) % N_DEV
            rdma = pltpu.make_async_remote_copy(
                src_ref=out_ref.at[pl.ds(c_send * m_c, m_c), :],
                dst_ref=out_ref.at[pl.ds(c_send * m_c, m_c), :],
                send_sem=send_sems.at[N_DEV - 1 + h],
                recv_sem=recv_sems.at[N_DEV - 1 + h],
                device_id=(right,),
                device_id_type=pl.DeviceIdType.MESH,
            )
            rdma.start()
            rdma.wait()

    return pl.pallas_call(
        body,
        out_shape=jax.ShapeDtypeStruct((m_per, n), jnp.float32),
        in_specs=[
            pl.BlockSpec(memory_space=pltpu.VMEM),
            pl.BlockSpec(memory_space=pltpu.VMEM),
        ],
        out_specs=pl.BlockSpec(memory_space=pltpu.VMEM),
        scratch_shapes=[
            pltpu.VMEM((m_c, k), jnp.float32),
            pltpu.VMEM((N_DEV - 1, m_c, k), jnp.float32),
            pltpu.SemaphoreType.DMA((2 * (N_DEV - 1),)),
            pltpu.SemaphoreType.DMA((2 * (N_DEV - 1),)),
        ],
        compiler_params=pltpu.CompilerParams(collective_id=0),
    )(t, W)


# baseline (device time: 311651 ns/iter reference)
import jax
import jax.numpy as jnp
from jax import lax
from jax.experimental import pallas as pl
from jax.experimental.pallas import tpu as pltpu

N_DEV = 4


def kernel(t, W):
    m_per, k = t.shape
    n = W.shape[1]
    m_c = m_per // N_DEV

    def body(t_ref, w_ref, out_ref, acc_ref, recv_ref, send_sems, recv_sems):
        d = lax.axis_index("i")
        left = (d - 1) % N_DEV
        right = (d + 1) % N_DEV

        barrier_sem = pltpu.get_barrier_semaphore()
        for nbr in (left, right):
            pl.semaphore_signal(
                barrier_sem, inc=1,
                device_id=(nbr,), device_id_type=pl.DeviceIdType.MESH,
            )
        pl.semaphore_wait(barrier_sem, 2)

        def t_chunk(c):
            return t_ref[pl.ds(c * m_c, m_c), :]

        for h in range(N_DEV - 1):
            c_send = (d - 1 - h) % N_DEV
            c_recv = (d - 2 - h) % N_DEV
            if h == 0:
                src = t_ref.at[pl.ds(c_send * m_c, m_c), :]
            else:
                src = acc_ref.at[:, :]
            rdma = pltpu.make_async_remote_copy(
                src_ref=src,
                dst_ref=recv_ref.at[h],
                send_sem=send_sems.at[h],
                recv_sem=recv_sems.at[h],
                device_id=(right,),
                device_id_type=pl.DeviceIdType.MESH,
            )
            rdma.start()
            rdma.wait()
            if h < N_DEV - 2:
                acc_ref[:, :] = recv_ref[h] + t_chunk(c_recv)

        s_chunk = recv_ref[N_DEV - 2] + t_chunk(d)

        y = lax.dot_general(
            s_chunk.astype(jnp.bfloat16),
            w_ref[:, :].astype(jnp.bfloat16),
            (((1,), (0,)), ((), ())),
            preferred_element_type=jnp.float32,
        )
        out_ref[pl.ds(d * m_c, m_c), :] = y

        for h in range(N_DEV - 1):
            c_send = (d - h) % N_DEV
            c_recv = (d - 1 - h) % N_DEV
            rdma = pltpu.make_async_remote_copy(
                src_ref=out_ref.at[pl.ds(c_send * m_c, m_c), :],
                dst_ref=out_ref.at[pl.ds(c_send * m_c, m_c), :],
                send_sem=send_sems.at[N_DEV - 1 + h],
                recv_sem=recv_sems.at[N_DEV - 1 + h],
                device_id=(right,),
                device_id_type=pl.DeviceIdType.MESH,
            )
            rdma.start()
            rdma.wait()

    return pl.pallas_call(
        body,
        out_shape=jax.ShapeDtypeStruct((m_per, n), jnp.float32),
        in_specs=[
            pl.BlockSpec(memory_space=pltpu.VMEM),
            pl.BlockSpec(memory_space=pltpu.VMEM),
        ],
        out_specs=pl.BlockSpec(memory_space=pltpu.VMEM),
        scratch_shapes=[
            pltpu.VMEM((m_c, k), jnp.float32),
            pltpu.VMEM((N_DEV - 1, m_c, k), jnp.float32),
            pltpu.SemaphoreType.DMA((2 * (N_DEV - 1),)),
            pltpu.SemaphoreType.DMA((2 * (N_DEV - 1),)),
        ],
        compiler_params=pltpu.CompilerParams(
            collective_id=0,
            vmem_limit_bytes=100 * 1024 * 1024,
        ),
    )(t, W)


# device time: 110462 ns/iter; 2.8213x vs baseline; 2.8213x over previous
import jax
import jax.numpy as jnp
from jax import lax
from jax.experimental import pallas as pl
from jax.experimental.pallas import tpu as pltpu

N_DEV = 4
N_HOP = N_DEV - 1


def kernel(t, W):
    m_per, k = t.shape
    n = W.shape[1]
    m_c = m_per // N_DEV
    m_h = m_c // 2

    def body(t_ref, w_ref, out_ref, send_ref, recv_ref, ag_ref,
             send_sems, recv_sems):
        d = lax.axis_index("i")
        left = (d - 1) % N_DEV
        right = (d + 1) % N_DEV

        barrier_sem = pltpu.get_barrier_semaphore()
        for nbr in (left, right):
            pl.semaphore_signal(
                barrier_sem, inc=1,
                device_id=(nbr,), device_id_type=pl.DeviceIdType.MESH,
            )
        pl.semaphore_wait(barrier_sem, 2)

        def tA(c):
            return t_ref[pl.ds(c * m_c, m_h), :]

        def tB(c):
            return t_ref[pl.ds(c * m_c + m_h, m_h), :]

        for h in range(N_HOP):
            cA_s = (d - 1 - h) % N_DEV
            cB_s = (d + 1 + h) % N_DEV
            if h == 0:
                send_ref[0, :, :] = tA(cA_s).astype(jnp.bfloat16)
                send_ref[1, :, :] = tB(cB_s).astype(jnp.bfloat16)
            else:
                send_ref[0, :, :] = (
                    recv_ref[0, h - 1].astype(jnp.float32) + tA(cA_s)
                ).astype(jnp.bfloat16)
                send_ref[1, :, :] = (
                    recv_ref[1, h - 1].astype(jnp.float32) + tB(cB_s)
                ).astype(jnp.bfloat16)
            rdmaA = pltpu.make_async_remote_copy(
                src_ref=send_ref.at[0],
                dst_ref=recv_ref.at[0, h],
                send_sem=send_sems.at[h],
                recv_sem=recv_sems.at[h],
                device_id=(right,),
                device_id_type=pl.DeviceIdType.MESH,
            )
            rdmaB = pltpu.make_async_remote_copy(
                src_ref=send_ref.at[1],
                dst_ref=recv_ref.at[1, h],
                send_sem=send_sems.at[N_HOP + h],
                recv_sem=recv_sems.at[N_HOP + h],
                device_id=(left,),
                device_id_type=pl.DeviceIdType.MESH,
            )
            rdmaA.start()
            rdmaB.start()
            rdmaA.wait()
            rdmaB.wait()

        w_bf = w_ref[:, :].astype(jnp.bfloat16)
        sA = (recv_ref[0, N_HOP - 1].astype(jnp.float32) + tA(d)).astype(
            jnp.bfloat16
        )
        yA = lax.dot_general(
            sA, w_bf, (((1,), (0,)), ((), ())),
            preferred_element_type=jnp.float32,
        )
        out_ref[pl.ds(d * m_c, m_h), :] = yA
        ag_ref[pl.ds(d * m_c, m_h), :] = yA.astype(jnp.bfloat16)

        sB = (recv_ref[1, N_HOP - 1].astype(jnp.float32) + tB(d)).astype(
            jnp.bfloat16
        )
        yB = lax.dot_general(
            sB, w_bf, (((1,), (0,)), ((), ())),
            preferred_element_type=jnp.float32,
        )
        out_ref[pl.ds(d * m_c + m_h, m_h), :] = yB
        ag_ref[pl.ds(d * m_c + m_h, m_h), :] = yB.astype(jnp.bfloat16)

        for h in range(N_HOP):
            cA_s = (d - h) % N_DEV
            cB_s = (d + h) % N_DEV
            cA_r = (d - 1 - h) % N_DEV
            cB_r = (d + 1 + h) % N_DEV
            rdmaA = pltpu.make_async_remote_copy(
                src_ref=ag_ref.at[pl.ds(cA_s * m_c, m_h), :],
                dst_ref=ag_ref.at[pl.ds(cA_s * m_c, m_h), :],
                send_sem=send_sems.at[2 * N_HOP + h],
                recv_sem=recv_sems.at[2 * N_HOP + h],
                device_id=(right,),
                device_id_type=pl.DeviceIdType.MESH,
            )
            rdmaB = pltpu.make_async_remote_copy(
                src_ref=ag_ref.at[pl.ds(cB_s * m_c + m_h, m_h), :],
                dst_ref=ag_ref.at[pl.ds(cB_s * m_c + m_h, m_h), :],
                send_sem=send_sems.at[3 * N_HOP + h],
                recv_sem=recv_sems.at[3 * N_HOP + h],
                device_id=(left,),
                device_id_type=pl.DeviceIdType.MESH,
            )
            rdmaA.start()
            rdmaB.start()
            rdmaA.wait()
            rdmaB.wait()
            out_ref[pl.ds(cA_r * m_c, m_h), :] = (
                ag_ref[pl.ds(cA_r * m_c, m_h), :].astype(jnp.float32)
            )
            out_ref[pl.ds(cB_r * m_c + m_h, m_h), :] = (
                ag_ref[pl.ds(cB_r * m_c + m_h, m_h), :].astype(jnp.float32)
            )

    return pl.pallas_call(
        body,
        out_shape=jax.ShapeDtypeStruct((m_per, n), jnp.float32),
        in_specs=[
            pl.BlockSpec(memory_space=pltpu.VMEM),
            pl.BlockSpec(memory_space=pltpu.VMEM),
        ],
        out_specs=pl.BlockSpec(memory_space=pltpu.VMEM),
        scratch_shapes=[
            pltpu.VMEM((2, m_h, k), jnp.bfloat16),
            pltpu.VMEM((2, N_HOP, m_h, k), jnp.bfloat16),
            pltpu.VMEM((m_per, n), jnp.bfloat16),
            pltpu.SemaphoreType.DMA((4 * N_HOP,)),
            pltpu.SemaphoreType.DMA((4 * N_HOP,)),
        ],
        compiler_params=pltpu.CompilerParams(
            collective_id=0,
            vmem_limit_bytes=100 * 1024 * 1024,
        ),
    )(t, W)


# device time: 101181 ns/iter; 3.0801x vs baseline; 1.0917x over previous
import jax
import jax.numpy as jnp
from jax import lax
from jax.experimental import pallas as pl
from jax.experimental.pallas import tpu as pltpu

N_DEV = 4
N_HOP = N_DEV - 1
N_SEG = 2
N_DIR = 2


def kernel(t, W):
    m_per, k = t.shape
    n = W.shape[1]
    m_c = m_per // N_DEV
    m_h = m_c // 2
    m_q = m_h // N_SEG

    n_sems_per_phase = N_DIR * N_HOP * N_SEG

    def body(t_ref, w_ref, out_ref, send_ref, recv_ref, ag_ref,
             send_sems, recv_sems):
        d = lax.axis_index("i")
        left = (d - 1) % N_DEV
        right = (d + 1) % N_DEV

        def row0(c, dirn, s):
            return c * m_c + dirn * m_h + s * m_q

        def t_seg(c, dirn, s):
            return t_ref[pl.ds(row0(c, dirn, s), m_q), :]

        def rs_c_send(dirn, h):
            return (d - 1 - h) % N_DEV if dirn == 0 else (d + 1 + h) % N_DEV

        def rs_c_recv(dirn, h):
            return (d - 2 - h) % N_DEV if dirn == 0 else (d + 2 + h) % N_DEV

        def ag_c_send(dirn, h):
            return (d - h) % N_DEV if dirn == 0 else (d + h) % N_DEV

        def ag_c_recv(dirn, h):
            return (d - 1 - h) % N_DEV if dirn == 0 else (d + 1 + h) % N_DEV

        def rs_desc(dirn, h, s):
            idx = dirn * (N_HOP * N_SEG) + h * N_SEG + s
            return pltpu.make_async_remote_copy(
                src_ref=send_ref.at[dirn, s],
                dst_ref=recv_ref.at[dirn, h, s],
                send_sem=send_sems.at[idx],
                recv_sem=recv_sems.at[idx],
                device_id=(right if dirn == 0 else left,),
                device_id_type=pl.DeviceIdType.MESH,
            )

        def ag_desc(dirn, h, s):
            idx = n_sems_per_phase + dirn * (N_HOP * N_SEG) + h * N_SEG + s
            rows = pl.ds(row0(ag_c_send(dirn, h), dirn, s), m_q)
            return pltpu.make_async_remote_copy(
                src_ref=ag_ref.at[rows, :],
                dst_ref=ag_ref.at[rows, :],
                send_sem=send_sems.at[idx],
                recv_sem=recv_sems.at[idx],
                device_id=(right if dirn == 0 else left,),
                device_id_type=pl.DeviceIdType.MESH,
            )

        barrier_sem = pltpu.get_barrier_semaphore()
        for nbr in (left, right):
            pl.semaphore_signal(
                barrier_sem, inc=1,
                device_id=(nbr,), device_id_type=pl.DeviceIdType.MESH,
            )
        for dirn in range(N_DIR):
            c = rs_c_send(dirn, 0)
            for s in range(N_SEG):
                send_ref[dirn, s] = t_seg(c, dirn, s).astype(jnp.bfloat16)
        pl.semaphore_wait(barrier_sem, 2)

        for dirn in range(N_DIR):
            for s in range(N_SEG):
                rs_desc(dirn, 0, s).start()

        w_bf = w_ref[:, :].astype(jnp.bfloat16)

        for h in range(1, N_HOP):
            for dirn in range(N_DIR):
                c = rs_c_send(dirn, h)
                for s in range(N_SEG):
                    prev = rs_desc(dirn, h - 1, s)
                    prev.wait_recv()
                    prev.wait_send()
                    send_ref[dirn, s] = (
                        recv_ref[dirn, h - 1, s].astype(jnp.float32)
                        + t_seg(c, dirn, s)
                    ).astype(jnp.bfloat16)
                    rs_desc(dirn, h, s).start()

        for dirn in range(N_DIR):
            for s in range(N_SEG):
                rs_desc(dirn, N_HOP - 1, s).wait_recv()
                s_seg = (
                    recv_ref[dirn, N_HOP - 1, s].astype(jnp.float32)
                    + t_seg(d, dirn, s)
                ).astype(jnp.bfloat16)
                y = lax.dot_general(
                    s_seg, w_bf, (((1,), (0,)), ((), ())),
                    preferred_element_type=jnp.float32,
                )
                rows = pl.ds(row0(d, dirn, s), m_q)
                out_ref[rows, :] = y
                ag_ref[rows, :] = y.astype(jnp.bfloat16)
                ag_desc(dirn, 0, s).start()

        for h in range(1, N_HOP):
            for dirn in range(N_DIR):
                for s in range(N_SEG):
                    ag_desc(dirn, h - 1, s).wait_recv()
                    ag_desc(dirn, h, s).start()
            for dirn in range(N_DIR):
                c = ag_c_recv(dirn, h - 1)
                for s in range(N_SEG):
                    rows = pl.ds(row0(c, dirn, s), m_q)
                    out_ref[rows, :] = ag_ref[rows, :].astype(jnp.float32)

        for dirn in range(N_DIR):
            c = ag_c_recv(dirn, N_HOP - 1)
            for s in range(N_SEG):
                ag_desc(dirn, N_HOP - 1, s).wait_recv()
                rows = pl.ds(row0(c, dirn, s), m_q)
                out_ref[rows, :] = ag_ref[rows, :].astype(jnp.float32)

        for dirn in range(N_DIR):
            for s in range(N_SEG):
                rs_desc(dirn, N_HOP - 1, s).wait_send()
        for h in range(N_HOP):
            for dirn in range(N_DIR):
                for s in range(N_SEG):
                    ag_desc(dirn, h, s).wait_send()

    n_sems = 2 * n_sems_per_phase
    return pl.pallas_call(
        body,
        out_shape=jax.ShapeDtypeStruct((m_per, n), jnp.float32),
        in_specs=[
            pl.BlockSpec(memory_space=pltpu.VMEM),
            pl.BlockSpec(memory_space=pltpu.VMEM),
        ],
        out_specs=pl.BlockSpec(memory_space=pltpu.VMEM),
        scratch_shapes=[
            pltpu.VMEM((N_DIR, N_SEG, m_q, k), jnp.bfloat16),
            pltpu.VMEM((N_DIR, N_HOP, N_SEG, m_q, k), jnp.bfloat16),
            pltpu.VMEM((m_per, n), jnp.bfloat16),
            pltpu.SemaphoreType.DMA((n_sems,)),
            pltpu.SemaphoreType.DMA((n_sems,)),
        ],
        compiler_params=pltpu.CompilerParams(
            collective_id=0,
            vmem_limit_bytes=100 * 1024 * 1024,
        ),
    )(t, W)
